# baseline (speedup 1.0000x reference)
import functools

import jax
import jax.numpy as jnp
from jax.experimental import pallas as pl
from jax.experimental.pallas import tpu as pltpu

B = 16; L = 512; C = 19; D = 256; H = 8; E = 8; NL = 2; DFF = 512; NC = 4
DH = D // H
PATCH = [8, 16, 32]
UPD = [19, 38]

NS_T = [L // p + 2 for p in PATCH]
NS_C = [u + 1 for u in UPD]
NP_T = 120
NP_C = 64


def _bounds(ns):
    out, s = [], 0
    for n in ns:
        out.append((s, s + n))
        s += n
    return out


def _ln(x):
    m = jnp.mean(x, axis=-1, keepdims=True)
    v = jnp.mean((x - m) ** 2, axis=-1, keepdims=True)
    return (x - m) * jax.lax.rsqrt(v + 1e-5)


def _layer_body(bounds, routers, np_, x_ref, gW_ref, gb_ref, Wq_ref, Wk_ref,
                Wv_ref, Wo_ref, W1_ref, b1_ref, W2_ref, b2_ref, out_ref):
    f32 = jnp.float32
    x = x_ref[0]

    xr = jnp.concatenate([x[p:p + 1, :] for p in routers], axis=0)
    gl = jnp.dot(xr, gW_ref[...], preferred_element_type=f32) + gb_ref[0]
    gl = gl - jnp.max(gl, axis=-1, keepdims=True)
    ge = jnp.exp(gl)
    gates = ge / jnp.sum(ge, axis=-1, keepdims=True)

    ri = jax.lax.broadcasted_iota(jnp.int32, (np_, np_), 0)
    ci = jax.lax.broadcasted_iota(jnp.int32, (np_, np_), 1)
    mask = jnp.zeros((np_, np_), jnp.bool_)
    for (s0, s1) in bounds:
        mask = mask | ((ri >= s0) & (ri < s1) & (ci >= s0) & (ci < s1))

    scale = 1.0 / (DH ** 0.5)
    acc = jnp.zeros((np_, D), f32)
    for e in range(E):
        q = jnp.dot(x, Wq_ref[e], preferred_element_type=f32)
        k = jnp.dot(x, Wk_ref[e], preferred_element_type=f32)
        v = jnp.dot(x, Wv_ref[e], preferred_element_type=f32)
        kT = k.T
        o_heads = []
        for h in range(H):
            qh = q[:, h * DH:(h + 1) * DH]
            khT = kT[h * DH:(h + 1) * DH, :]
            s = jnp.dot(qh, khT, preferred_element_type=f32) * scale
            s = jnp.where(mask, s, -1e30)
            s = s - jnp.max(s, axis=-1, keepdims=True)
            p_ = jnp.exp(s)
            a = p_ / jnp.sum(p_, axis=-1, keepdims=True)
            vh = v[:, h * DH:(h + 1) * DH]
            o_heads.append(jnp.dot(a, vh, preferred_element_type=f32))
        o = jnp.concatenate(o_heads, axis=1)
        oe = jnp.dot(o, Wo_ref[e], preferred_element_type=f32)
        gparts = []
        for si, (s0, s1) in enumerate(bounds):
            gparts.append(jnp.broadcast_to(gates[si:si + 1, e:e + 1],
                                           (s1 - s0, 1)))
        tail = np_ - bounds[-1][1]
        if tail:
            gparts.append(jnp.zeros((tail, 1), f32))
        gcol = jnp.concatenate(gparts, axis=0)
        acc = acc + gcol * oe

    x1 = _ln(x + acc)
    h1 = jax.nn.gelu(jnp.dot(x1, W1_ref[...], preferred_element_type=f32)
                     + b1_ref[0])
    x2 = _ln(x1 + jnp.dot(h1, W2_ref[...], preferred_element_type=f32)
             + b2_ref[0])
    out_ref[0] = x2


@functools.partial(jax.jit, static_argnums=(0,))
def _layer(which, x, gW, gb, Wq, Wk, Wv, Wo, W1, b1, W2, b2):
    ns = NS_T if which == 't' else NS_C
    np_ = NP_T if which == 't' else NP_C
    bounds = _bounds(ns)
    routers = [b_ - 1 for (_, b_) in bounds]
    body = functools.partial(_layer_body, bounds, routers, np_)
    return pl.pallas_call(
        body,
        grid=(B,),
        in_specs=[
            pl.BlockSpec((1, np_, D), lambda b: (b, 0, 0)),
            pl.BlockSpec((D, E), lambda b: (0, 0)),
            pl.BlockSpec((1, E), lambda b: (0, 0)),
            pl.BlockSpec((E, D, D), lambda b: (0, 0, 0)),
            pl.BlockSpec((E, D, D), lambda b: (0, 0, 0)),
            pl.BlockSpec((E, D, D), lambda b: (0, 0, 0)),
            pl.BlockSpec((E, D, D), lambda b: (0, 0, 0)),
            pl.BlockSpec((D, DFF), lambda b: (0, 0)),
            pl.BlockSpec((1, DFF), lambda b: (0, 0)),
            pl.BlockSpec((DFF, D), lambda b: (0, 0)),
            pl.BlockSpec((1, D), lambda b: (0, 0)),
        ],
        out_specs=pl.BlockSpec((1, np_, D), lambda b: (b, 0, 0)),
        out_shape=jax.ShapeDtypeStruct((B, np_, D), jnp.float32),
    )(x, gW, gb, Wq, Wk, Wv, Wo, W1, b1, W2, b2)


def kernel(x_enc, x_mark_enc, x_dec, x_mark_dec, params):
    p = params
    f32 = jnp.float32

    xs_t = []
    for gi, patch in enumerate(PATCH):
        pad = jnp.repeat(x_enc[:, -1:, :], patch, axis=1)
        xp = jnp.concatenate([x_enc, pad], axis=1)
        n = xp.shape[1] // patch
        tok = xp.reshape(B, n, patch * C) @ p['emb_t_W'][gi] + p['emb_t_b'][gi]
        r = jnp.broadcast_to(p['router_t'][gi][None, None, :], (B, 1, D))
        xs_t.append(jnp.concatenate([tok, r], axis=1) + p['pos_t'][gi][None])
    xs_c = []
    for si, u in enumerate(UPD):
        xc = jnp.einsum('blc,cu->bul', x_enc, p['up_W'][si])
        tok = xc @ p['emb_c_W'][si] + p['emb_c_b'][si]
        r = jnp.broadcast_to(p['router_c'][si][None, None, :], (B, 1, D))
        xs_c.append(jnp.concatenate([tok, r], axis=1) + p['pos_c'][si][None])

    Xt = jnp.concatenate(
        xs_t + [jnp.zeros((B, NP_T - sum(NS_T), D), f32)], axis=1)
    Xc = jnp.concatenate(
        xs_c + [jnp.zeros((B, NP_C - sum(NS_C), D), f32)], axis=1)

    gb_t = p['gate_b_t'].reshape(1, E)
    gb_c = p['gate_b_c'].reshape(1, E)

    for l in range(NL):
        W1 = p['ffn_W1'][l]; b1 = p['ffn_b1'][l].reshape(1, DFF)
        W2 = p['ffn_W2'][l]; b2 = p['ffn_b2'][l].reshape(1, D)
        Xt = _layer('t', Xt, p['gate_W_t'], gb_t, p['Wq_t'], p['Wk_t'],
                    p['Wv_t'], p['Wo_t'], W1, b1, W2, b2)
        Xc = _layer('c', Xc, p['gate_W_c'], gb_c, p['Wq_c'], p['Wk_c'],
                    p['Wv_c'], p['Wo_c'], W1, b1, W2, b2)

    rt = [b_ - 1 for (_, b_) in _bounds(NS_T)]
    rc = [b_ - 1 for (_, b_) in _bounds(NS_C)]
    t_repr = jnp.mean(_ln(Xt[:, jnp.array(rt), :]), axis=1)
    c_repr = jnp.mean(_ln(Xc[:, jnp.array(rc), :]), axis=1)
    final = jnp.concatenate([t_repr, c_repr], axis=1)
    return jax.nn.gelu(final) @ p['clf_W'] + p['clf_b']


# speedup vs baseline: 2.2880x; 2.2880x over previous
import functools

import jax
import jax.numpy as jnp
from jax.experimental import pallas as pl
from jax.experimental.pallas import tpu as pltpu

B = 16; L = 512; C = 19; D = 256; H = 8; E = 8; NL = 2; DFF = 512; NC = 4
DH = D // H
ED = E * D
PATCH = [8, 16, 32]
UPD = [19, 38]

NS_T = [L // p + 2 for p in PATCH]
NS_C = [u + 1 for u in UPD]
NP_T = 120
NP_C = 64

BF = jnp.bfloat16
F32 = jnp.float32


def _bounds(ns):
    out, s = [], 0
    for n in ns:
        out.append((s, s + n))
        s += n
    return out


def _ln(x):
    m = jnp.mean(x, axis=-1, keepdims=True)
    v = jnp.mean((x - m) ** 2, axis=-1, keepdims=True)
    return (x - m) * jax.lax.rsqrt(v + 1e-5)


def _layer_body(bounds, routers, np_, x_ref, gW_ref, gb_ref, WqA_ref,
                WkAT_ref, WvA_ref, WoS_ref, W1_ref, b1_ref, W2_ref, b2_ref,
                out_ref):
    x = x_ref[0]
    xb = x.astype(BF)
    xT = x.T.astype(BF)

    xr = jnp.concatenate([x[p:p + 1, :] for p in routers], axis=0)
    gl = jnp.dot(xr, gW_ref[...], preferred_element_type=F32) + gb_ref[0]
    gl = gl - jnp.max(gl, axis=-1, keepdims=True)
    ge = jnp.exp(gl)
    gates = ge / jnp.sum(ge, axis=-1, keepdims=True)
    gparts = []
    for si, (s0, s1) in enumerate(bounds):
        gparts.append(jnp.broadcast_to(gates[si:si + 1, :], (s1 - s0, E)))
    tail = np_ - bounds[-1][1]
    if tail:
        gparts.append(jnp.zeros((tail, E), F32))
    G = jnp.concatenate(gparts, axis=0)

    ri = jax.lax.broadcasted_iota(jnp.int32, (np_, np_), 0)
    ci = jax.lax.broadcasted_iota(jnp.int32, (np_, np_), 1)
    mask = jnp.zeros((np_, np_), jnp.bool_)
    for (s0, s1) in bounds:
        mask = mask | ((ri >= s0) & (ri < s1) & (ci >= s0) & (ci < s1))
    bias = jnp.where(mask, 0.0, -1e30).astype(F32)

    Q = jnp.dot(xb, WqA_ref[...],
                preferred_element_type=F32).astype(BF)
    KT = jnp.dot(WkAT_ref[...], xT,
                 preferred_element_type=F32).astype(BF)
    V = jnp.dot(xb, WvA_ref[...],
                preferred_element_type=F32).astype(BF)

    scale = 1.0 / (DH ** 0.5)
    og_parts = []
    for e in range(E):
        o_heads = []
        for h in range(H):
            base = e * D + h * DH
            s = jnp.dot(Q[:, base:base + DH], KT[base:base + DH, :],
                        preferred_element_type=F32)
            p_ = jnp.exp(s * scale + bias)
            r = jax.lax.rsqrt(jnp.square(jnp.sum(p_, axis=-1,
                                                 keepdims=True)) + 1e-30)
            o_h = jnp.dot(p_.astype(BF), V[:, base:base + DH],
                          preferred_element_type=F32)
            o_heads.append(o_h * r)
        o = jnp.concatenate(o_heads, axis=1)
        og_parts.append(o * G[:, e:e + 1])
    OG = jnp.concatenate(og_parts, axis=1).astype(BF)
    acc = jnp.dot(OG, WoS_ref[...], preferred_element_type=F32)

    x1 = _ln(x + acc)
    h1 = jax.nn.gelu(jnp.dot(x1.astype(BF), W1_ref[...],
                             preferred_element_type=F32) + b1_ref[0])
    x2 = _ln(x1 + jnp.dot(h1.astype(BF), W2_ref[...],
                          preferred_element_type=F32) + b2_ref[0])
    out_ref[0] = x2


@functools.partial(jax.jit, static_argnums=(0,))
def _layer(which, x, gW, gb, WqA, WkAT, WvA, WoS, W1, b1, W2, b2):
    ns = NS_T if which == 't' else NS_C
    np_ = NP_T if which == 't' else NP_C
    bounds = _bounds(ns)
    routers = [b_ - 1 for (_, b_) in bounds]
    body = functools.partial(_layer_body, bounds, routers, np_)
    return pl.pallas_call(
        body,
        grid=(B,),
        in_specs=[
            pl.BlockSpec((1, np_, D), lambda b: (b, 0, 0)),
            pl.BlockSpec((D, E), lambda b: (0, 0)),
            pl.BlockSpec((1, E), lambda b: (0, 0)),
            pl.BlockSpec((D, ED), lambda b: (0, 0)),
            pl.BlockSpec((ED, D), lambda b: (0, 0)),
            pl.BlockSpec((D, ED), lambda b: (0, 0)),
            pl.BlockSpec((ED, D), lambda b: (0, 0)),
            pl.BlockSpec((D, DFF), lambda b: (0, 0)),
            pl.BlockSpec((1, DFF), lambda b: (0, 0)),
            pl.BlockSpec((DFF, D), lambda b: (0, 0)),
            pl.BlockSpec((1, D), lambda b: (0, 0)),
        ],
        out_specs=pl.BlockSpec((1, np_, D), lambda b: (b, 0, 0)),
        out_shape=jax.ShapeDtypeStruct((B, np_, D), jnp.float32),
    )(x, gW, gb, WqA, WkAT, WvA, WoS, W1, b1, W2, b2)


def _pack_branch(p, br):
    Wq = p['Wq_' + br]; Wk = p['Wk_' + br]; Wv = p['Wv_' + br]
    Wo = p['Wo_' + br]
    WqA = jnp.transpose(Wq, (1, 0, 2)).reshape(D, ED).astype(BF)
    WkAT = jnp.transpose(Wk, (0, 2, 1)).reshape(ED, D).astype(BF)
    WvA = jnp.transpose(Wv, (1, 0, 2)).reshape(D, ED).astype(BF)
    WoS = Wo.reshape(ED, D).astype(BF)
    return WqA, WkAT, WvA, WoS


def kernel(x_enc, x_mark_enc, x_dec, x_mark_dec, params):
    p = params

    xs_t = []
    for gi, patch in enumerate(PATCH):
        pad = jnp.repeat(x_enc[:, -1:, :], patch, axis=1)
        xp = jnp.concatenate([x_enc, pad], axis=1)
        n = xp.shape[1] // patch
        tok = xp.reshape(B, n, patch * C) @ p['emb_t_W'][gi] + p['emb_t_b'][gi]
        r = jnp.broadcast_to(p['router_t'][gi][None, None, :], (B, 1, D))
        xs_t.append(jnp.concatenate([tok, r], axis=1) + p['pos_t'][gi][None])
    xs_c = []
    for si, u in enumerate(UPD):
        xc = jnp.einsum('blc,cu->bul', x_enc, p['up_W'][si])
        tok = xc @ p['emb_c_W'][si] + p['emb_c_b'][si]
        r = jnp.broadcast_to(p['router_c'][si][None, None, :], (B, 1, D))
        xs_c.append(jnp.concatenate([tok, r], axis=1) + p['pos_c'][si][None])

    Xt = jnp.concatenate(
        xs_t + [jnp.zeros((B, NP_T - sum(NS_T), D), F32)], axis=1)
    Xc = jnp.concatenate(
        xs_c + [jnp.zeros((B, NP_C - sum(NS_C), D), F32)], axis=1)

    gb_t = p['gate_b_t'].reshape(1, E)
    gb_c = p['gate_b_c'].reshape(1, E)
    packed_t = _pack_branch(p, 't')
    packed_c = _pack_branch(p, 'c')

    for l in range(NL):
        W1 = p['ffn_W1'][l].astype(BF); b1 = p['ffn_b1'][l].reshape(1, DFF)
        W2 = p['ffn_W2'][l].astype(BF); b2 = p['ffn_b2'][l].reshape(1, D)
        Xt = _layer('t', Xt, p['gate_W_t'], gb_t, *packed_t, W1, b1, W2, b2)
        Xc = _layer('c', Xc, p['gate_W_c'], gb_c, *packed_c, W1, b1, W2, b2)

    rt = [b_ - 1 for (_, b_) in _bounds(NS_T)]
    rc = [b_ - 1 for (_, b_) in _bounds(NS_C)]
    t_repr = jnp.mean(_ln(Xt[:, jnp.array(rt), :]), axis=1)
    c_repr = jnp.mean(_ln(Xc[:, jnp.array(rc), :]), axis=1)
    final = jnp.concatenate([t_repr, c_repr], axis=1)
    return jax.nn.gelu(final) @ p['clf_W'] + p['clf_b']


# speedup vs baseline: 3.2007x; 1.3989x over previous
import functools

import jax
import jax.numpy as jnp
from jax.experimental import pallas as pl
from jax.experimental.pallas import tpu as pltpu

B = 16; L = 512; C = 19; D = 256; H = 8; E = 8; NL = 2; DFF = 512; NC = 4
DH = D // H
ED = E * D
PATCH = [8, 16, 32]
UPD = [19, 38]

NS_T = [L // p + 2 for p in PATCH]
NS_C = [u + 1 for u in UPD]
NP_T = 120
NP_C = 64

BF = jnp.bfloat16
F32 = jnp.float32


def _bounds(ns):
    out, s = [], 0
    for n in ns:
        out.append((s, s + n))
        s += n
    return out


def _ln(x):
    m = jnp.mean(x, axis=-1, keepdims=True)
    v = jnp.mean((x - m) ** 2, axis=-1, keepdims=True)
    return (x - m) * jax.lax.rsqrt(v + 1e-5)


def _layer_body(bounds, routers, np_, x_ref, gW_ref, gb_ref, WqA_ref,
                WkAT_ref, WvA_ref, WoS_ref, W1_ref, b1_ref, W2_ref, b2_ref,
                out_ref):
    x = x_ref[0]
    xb = x.astype(BF)
    xT = x.T.astype(BF)

    xr = jnp.concatenate([x[p:p + 1, :] for p in routers], axis=0)
    gl = jnp.dot(xr, gW_ref[...], preferred_element_type=F32) + gb_ref[0]
    gl = gl - jnp.max(gl, axis=-1, keepdims=True)
    ge = jnp.exp(gl)
    gates = ge / jnp.sum(ge, axis=-1, keepdims=True)
    gparts = []
    for si, (s0, s1) in enumerate(bounds):
        gparts.append(jnp.broadcast_to(gates[si:si + 1, :], (s1 - s0, E)))
    tail = np_ - bounds[-1][1]
    if tail:
        gparts.append(jnp.zeros((tail, E), F32))
    G = jnp.concatenate(gparts, axis=0)

    ri = jax.lax.broadcasted_iota(jnp.int32, (np_, np_), 0)
    ci = jax.lax.broadcasted_iota(jnp.int32, (np_, np_), 1)
    mask = jnp.zeros((np_, np_), jnp.bool_)
    for (s0, s1) in bounds:
        mask = mask | ((ri >= s0) & (ri < s1) & (ci >= s0) & (ci < s1))
    mask01 = mask.astype(F32)
    hi = jax.lax.broadcasted_iota(jnp.int32, (H * np_, H), 0) // np_
    hj = jax.lax.broadcasted_iota(jnp.int32, (H * np_, H), 1)
    ones_bd = (hi == hj).astype(BF)

    Q = jnp.dot(xb, WqA_ref[...],
                preferred_element_type=F32).astype(BF)
    KT = jnp.dot(WkAT_ref[...], xT,
                 preferred_element_type=F32).astype(BF)
    V = jnp.dot(xb, WvA_ref[...],
                preferred_element_type=F32).astype(BF)

    og_parts = []
    for e in range(E):
        p_heads = []
        for h in range(H):
            base = e * D + h * DH
            s = jnp.dot(Q[:, base:base + DH], KT[base:base + DH, :],
                        preferred_element_type=F32)
            p_heads.append((jnp.exp(s) * mask01).astype(BF))
        P = jnp.concatenate(p_heads, axis=1)
        rs = jnp.dot(P, ones_bd, preferred_element_type=F32)
        rr = 1.0 / (rs + 1e-30)
        o_heads = []
        for h in range(H):
            base = e * D + h * DH
            o_h = jnp.dot(p_heads[h], V[:, base:base + DH],
                          preferred_element_type=F32)
            o_heads.append(o_h * rr[:, h:h + 1])
        o = jnp.concatenate(o_heads, axis=1)
        og_parts.append(o * G[:, e:e + 1])
    OG = jnp.concatenate(og_parts, axis=1).astype(BF)
    acc = jnp.dot(OG, WoS_ref[...], preferred_element_type=F32)

    x1 = _ln(x + acc)
    h1 = jax.nn.gelu(jnp.dot(x1.astype(BF), W1_ref[...],
                             preferred_element_type=F32) + b1_ref[0])
    x2 = _ln(x1 + jnp.dot(h1.astype(BF), W2_ref[...],
                          preferred_element_type=F32) + b2_ref[0])
    out_ref[0] = x2


@functools.partial(jax.jit, static_argnums=(0,))
def _layer(which, x, gW, gb, WqA, WkAT, WvA, WoS, W1, b1, W2, b2):
    ns = NS_T if which == 't' else NS_C
    np_ = NP_T if which == 't' else NP_C
    bounds = _bounds(ns)
    routers = [b_ - 1 for (_, b_) in bounds]
    body = functools.partial(_layer_body, bounds, routers, np_)
    return pl.pallas_call(
        body,
        grid=(B,),
        in_specs=[
            pl.BlockSpec((1, np_, D), lambda b: (b, 0, 0)),
            pl.BlockSpec((D, E), lambda b: (0, 0)),
            pl.BlockSpec((1, E), lambda b: (0, 0)),
            pl.BlockSpec((D, ED), lambda b: (0, 0)),
            pl.BlockSpec((ED, D), lambda b: (0, 0)),
            pl.BlockSpec((D, ED), lambda b: (0, 0)),
            pl.BlockSpec((ED, D), lambda b: (0, 0)),
            pl.BlockSpec((D, DFF), lambda b: (0, 0)),
            pl.BlockSpec((1, DFF), lambda b: (0, 0)),
            pl.BlockSpec((DFF, D), lambda b: (0, 0)),
            pl.BlockSpec((1, D), lambda b: (0, 0)),
        ],
        out_specs=pl.BlockSpec((1, np_, D), lambda b: (b, 0, 0)),
        out_shape=jax.ShapeDtypeStruct((B, np_, D), jnp.float32),
    )(x, gW, gb, WqA, WkAT, WvA, WoS, W1, b1, W2, b2)


def _pack_branch(p, br):
    Wq = p['Wq_' + br]; Wk = p['Wk_' + br]; Wv = p['Wv_' + br]
    Wo = p['Wo_' + br]
    scale = 1.0 / (DH ** 0.5)
    WqA = (jnp.transpose(Wq, (1, 0, 2)).reshape(D, ED) * scale).astype(BF)
    WkAT = jnp.transpose(Wk, (0, 2, 1)).reshape(ED, D).astype(BF)
    WvA = jnp.transpose(Wv, (1, 0, 2)).reshape(D, ED).astype(BF)
    WoS = Wo.reshape(ED, D).astype(BF)
    return WqA, WkAT, WvA, WoS


def kernel(x_enc, x_mark_enc, x_dec, x_mark_dec, params):
    p = params

    xs_t = []
    for gi, patch in enumerate(PATCH):
        pad = jnp.repeat(x_enc[:, -1:, :], patch, axis=1)
        xp = jnp.concatenate([x_enc, pad], axis=1)
        n = xp.shape[1] // patch
        tok = xp.reshape(B, n, patch * C) @ p['emb_t_W'][gi] + p['emb_t_b'][gi]
        r = jnp.broadcast_to(p['router_t'][gi][None, None, :], (B, 1, D))
        xs_t.append(jnp.concatenate([tok, r], axis=1) + p['pos_t'][gi][None])
    xs_c = []
    for si, u in enumerate(UPD):
        xc = jnp.einsum('blc,cu->bul', x_enc, p['up_W'][si])
        tok = xc @ p['emb_c_W'][si] + p['emb_c_b'][si]
        r = jnp.broadcast_to(p['router_c'][si][None, None, :], (B, 1, D))
        xs_c.append(jnp.concatenate([tok, r], axis=1) + p['pos_c'][si][None])

    Xt = jnp.concatenate(
        xs_t + [jnp.zeros((B, NP_T - sum(NS_T), D), F32)], axis=1)
    Xc = jnp.concatenate(
        xs_c + [jnp.zeros((B, NP_C - sum(NS_C), D), F32)], axis=1)

    gb_t = p['gate_b_t'].reshape(1, E)
    gb_c = p['gate_b_c'].reshape(1, E)
    packed_t = _pack_branch(p, 't')
    packed_c = _pack_branch(p, 'c')

    for l in range(NL):
        W1 = p['ffn_W1'][l].astype(BF); b1 = p['ffn_b1'][l].reshape(1, DFF)
        W2 = p['ffn_W2'][l].astype(BF); b2 = p['ffn_b2'][l].reshape(1, D)
        Xt = _layer('t', Xt, p['gate_W_t'], gb_t, *packed_t, W1, b1, W2, b2)
        Xc = _layer('c', Xc, p['gate_W_c'], gb_c, *packed_c, W1, b1, W2, b2)

    rt = [b_ - 1 for (_, b_) in _bounds(NS_T)]
    rc = [b_ - 1 for (_, b_) in _bounds(NS_C)]
    t_repr = jnp.mean(_ln(Xt[:, jnp.array(rt), :]), axis=1)
    c_repr = jnp.mean(_ln(Xc[:, jnp.array(rc), :]), axis=1)
    final = jnp.concatenate([t_repr, c_repr], axis=1)
    return jax.nn.gelu(final) @ p['clf_W'] + p['clf_b']


# speedup vs baseline: 3.3025x; 1.0318x over previous
import functools

import jax
import jax.numpy as jnp
from jax.experimental import pallas as pl
from jax.experimental.pallas import tpu as pltpu

B = 16; L = 512; C = 19; D = 256; H = 8; E = 8; NL = 2; DFF = 512; NC = 4
DH = D // H
ED = E * D
PATCH = [8, 16, 32]
UPD = [19, 38]

NS_T = [L // p + 2 for p in PATCH]
NS_C = [u + 1 for u in UPD]
NP_T = 120
NP_C = 64

BF = jnp.bfloat16
F32 = jnp.float32


def _bounds(ns):
    out, s = [], 0
    for n in ns:
        out.append((s, s + n))
        s += n
    return out


def _ln(x):
    m = jnp.mean(x, axis=-1, keepdims=True)
    v = jnp.mean((x - m) ** 2, axis=-1, keepdims=True)
    return (x - m) * jax.lax.rsqrt(v + 1e-5)


def _layer_body(bounds, routers, np_, x_ref, gW_ref, gb_ref, WqA_ref,
                WkAT_ref, WvA_ref, WoS_ref, W1_ref, b1_ref, W2_ref, b2_ref,
                out_ref):
    x = x_ref[0]
    xb = x.astype(BF)
    xT = x.T.astype(BF)

    xr = jnp.concatenate([x[p:p + 1, :] for p in routers], axis=0)
    gl = jnp.dot(xr, gW_ref[...], preferred_element_type=F32) + gb_ref[0]
    gl = gl - jnp.max(gl, axis=-1, keepdims=True)
    ge = jnp.exp(gl)
    gates = ge / jnp.sum(ge, axis=-1, keepdims=True)
    gparts = []
    for si, (s0, s1) in enumerate(bounds):
        gparts.append(jnp.broadcast_to(gates[si:si + 1, :], (s1 - s0, E)))
    tail = np_ - bounds[-1][1]
    if tail:
        gparts.append(jnp.zeros((tail, E), F32))
    G = jnp.concatenate(gparts, axis=0)

    ri = jax.lax.broadcasted_iota(jnp.int32, (np_, np_), 0)
    ci = jax.lax.broadcasted_iota(jnp.int32, (np_, np_), 1)
    mask = jnp.zeros((np_, np_), jnp.bool_)
    for (s0, s1) in bounds:
        mask = mask | ((ri >= s0) & (ri < s1) & (ci >= s0) & (ci < s1))
    mask01 = mask.astype(F32)
    hi = jax.lax.broadcasted_iota(jnp.int32, (H * np_, H), 0) // np_
    hj = jax.lax.broadcasted_iota(jnp.int32, (H * np_, H), 1)
    ones_bd = (hi == hj).astype(BF)
    xi = jax.lax.broadcasted_iota(jnp.int32, (H, D), 0)
    xj = jax.lax.broadcasted_iota(jnp.int32, (H, D), 1) // DH
    exp8 = (xi == xj).astype(F32)

    Q = jnp.dot(xb, WqA_ref[...],
                preferred_element_type=F32).astype(BF)
    KT = jnp.dot(WkAT_ref[...], xT,
                 preferred_element_type=F32).astype(BF)
    V = jnp.dot(xb, WvA_ref[...],
                preferred_element_type=F32).astype(BF)

    og_parts = []
    for e in range(E):
        p_heads = []
        for h in range(H):
            base = e * D + h * DH
            s = jnp.dot(Q[:, base:base + DH], KT[base:base + DH, :],
                        preferred_element_type=F32)
            p_heads.append((jnp.exp(s) * mask01).astype(BF))
        P = jnp.concatenate(p_heads, axis=1)
        rs = jnp.dot(P, ones_bd, preferred_element_type=F32)
        rrg = G[:, e:e + 1] / (rs + 1e-30)
        scale_e = jnp.dot(rrg, exp8, preferred_element_type=F32)
        o_heads = []
        for h in range(H):
            base = e * D + h * DH
            o_heads.append(jnp.dot(p_heads[h], V[:, base:base + DH],
                                   preferred_element_type=F32))
        o = jnp.concatenate(o_heads, axis=1)
        og_parts.append(o * scale_e)
    OG = jnp.concatenate(og_parts, axis=1).astype(BF)
    acc = jnp.dot(OG, WoS_ref[...], preferred_element_type=F32)

    x1 = _ln(x + acc)
    h1 = jax.nn.gelu(jnp.dot(x1.astype(BF), W1_ref[...],
                             preferred_element_type=F32) + b1_ref[0])
    x2 = _ln(x1 + jnp.dot(h1.astype(BF), W2_ref[...],
                          preferred_element_type=F32) + b2_ref[0])
    out_ref[0] = x2


@functools.partial(jax.jit, static_argnums=(0,))
def _layer(which, x, gW, gb, WqA, WkAT, WvA, WoS, W1, b1, W2, b2):
    ns = NS_T if which == 't' else NS_C
    np_ = NP_T if which == 't' else NP_C
    bounds = _bounds(ns)
    routers = [b_ - 1 for (_, b_) in bounds]
    body = functools.partial(_layer_body, bounds, routers, np_)
    return pl.pallas_call(
        body,
        grid=(B,),
        in_specs=[
            pl.BlockSpec((1, np_, D), lambda b: (b, 0, 0)),
            pl.BlockSpec((D, E), lambda b: (0, 0)),
            pl.BlockSpec((1, E), lambda b: (0, 0)),
            pl.BlockSpec((D, ED), lambda b: (0, 0)),
            pl.BlockSpec((ED, D), lambda b: (0, 0)),
            pl.BlockSpec((D, ED), lambda b: (0, 0)),
            pl.BlockSpec((ED, D), lambda b: (0, 0)),
            pl.BlockSpec((D, DFF), lambda b: (0, 0)),
            pl.BlockSpec((1, DFF), lambda b: (0, 0)),
            pl.BlockSpec((DFF, D), lambda b: (0, 0)),
            pl.BlockSpec((1, D), lambda b: (0, 0)),
        ],
        out_specs=pl.BlockSpec((1, np_, D), lambda b: (b, 0, 0)),
        out_shape=jax.ShapeDtypeStruct((B, np_, D), jnp.float32),
    )(x, gW, gb, WqA, WkAT, WvA, WoS, W1, b1, W2, b2)


def _pack_branch(p, br):
    Wq = p['Wq_' + br]; Wk = p['Wk_' + br]; Wv = p['Wv_' + br]
    Wo = p['Wo_' + br]
    scale = 1.0 / (DH ** 0.5)
    WqA = (jnp.transpose(Wq, (1, 0, 2)).reshape(D, ED) * scale).astype(BF)
    WkAT = jnp.transpose(Wk, (0, 2, 1)).reshape(ED, D).astype(BF)
    WvA = jnp.transpose(Wv, (1, 0, 2)).reshape(D, ED).astype(BF)
    WoS = Wo.reshape(ED, D).astype(BF)
    return WqA, WkAT, WvA, WoS


def kernel(x_enc, x_mark_enc, x_dec, x_mark_dec, params):
    p = params

    xs_t = []
    for gi, patch in enumerate(PATCH):
        pad = jnp.repeat(x_enc[:, -1:, :], patch, axis=1)
        xp = jnp.concatenate([x_enc, pad], axis=1)
        n = xp.shape[1] // patch
        tok = xp.reshape(B, n, patch * C) @ p['emb_t_W'][gi] + p['emb_t_b'][gi]
        r = jnp.broadcast_to(p['router_t'][gi][None, None, :], (B, 1, D))
        xs_t.append(jnp.concatenate([tok, r], axis=1) + p['pos_t'][gi][None])
    xs_c = []
    for si, u in enumerate(UPD):
        xc = jnp.einsum('blc,cu->bul', x_enc, p['up_W'][si])
        tok = xc @ p['emb_c_W'][si] + p['emb_c_b'][si]
        r = jnp.broadcast_to(p['router_c'][si][None, None, :], (B, 1, D))
        xs_c.append(jnp.concatenate([tok, r], axis=1) + p['pos_c'][si][None])

    Xt = jnp.concatenate(
        xs_t + [jnp.zeros((B, NP_T - sum(NS_T), D), F32)], axis=1)
    Xc = jnp.concatenate(
        xs_c + [jnp.zeros((B, NP_C - sum(NS_C), D), F32)], axis=1)

    gb_t = p['gate_b_t'].reshape(1, E)
    gb_c = p['gate_b_c'].reshape(1, E)
    packed_t = _pack_branch(p, 't')
    packed_c = _pack_branch(p, 'c')

    for l in range(NL):
        W1 = p['ffn_W1'][l].astype(BF); b1 = p['ffn_b1'][l].reshape(1, DFF)
        W2 = p['ffn_W2'][l].astype(BF); b2 = p['ffn_b2'][l].reshape(1, D)
        Xt = _layer('t', Xt, p['gate_W_t'], gb_t, *packed_t, W1, b1, W2, b2)
        Xc = _layer('c', Xc, p['gate_W_c'], gb_c, *packed_c, W1, b1, W2, b2)

    rt = [b_ - 1 for (_, b_) in _bounds(NS_T)]
    rc = [b_ - 1 for (_, b_) in _bounds(NS_C)]
    t_repr = jnp.mean(_ln(Xt[:, jnp.array(rt), :]), axis=1)
    c_repr = jnp.mean(_ln(Xc[:, jnp.array(rc), :]), axis=1)
    final = jnp.concatenate([t_repr, c_repr], axis=1)
    return jax.nn.gelu(final) @ p['clf_W'] + p['clf_b']


# speedup vs baseline: 3.4984x; 1.0593x over previous
import functools

import jax
import jax.numpy as jnp
from jax.experimental import pallas as pl
from jax.experimental.pallas import tpu as pltpu

B = 16; L = 512; C = 19; D = 256; H = 8; E = 8; NL = 2; DFF = 512; NC = 4
DH = D // H
ED = E * D
PATCH = [8, 16, 32]
UPD = [19, 38]

NS_T = [L // p + 2 for p in PATCH]
NS_C = [u + 1 for u in UPD]
NP_T = 120
NP_C = 64

BF = jnp.bfloat16
F32 = jnp.float32


def _bounds(ns):
    out, s = [], 0
    for n in ns:
        out.append((s, s + n))
        s += n
    return out


BOUNDS_T = _bounds(NS_T)
BOUNDS_C = _bounds(NS_C)
ROUTERS_T = [b - 1 for (_, b) in BOUNDS_T]
ROUTERS_C = [b - 1 for (_, b) in BOUNDS_C]


def _ln(x):
    m = jnp.mean(x, axis=-1, keepdims=True)
    v = jnp.mean((x - m) ** 2, axis=-1, keepdims=True)
    return (x - m) * jax.lax.rsqrt(v + 1e-5)


def _masks(bounds, np_):
    ri = jax.lax.broadcasted_iota(jnp.int32, (np_, np_), 0)
    ci = jax.lax.broadcasted_iota(jnp.int32, (np_, np_), 1)
    mask = jnp.zeros((np_, np_), jnp.bool_)
    for (s0, s1) in bounds:
        mask = mask | ((ri >= s0) & (ri < s1) & (ci >= s0) & (ci < s1))
    mask01 = mask.astype(F32)
    hi = jax.lax.broadcasted_iota(jnp.int32, (H * np_, H), 0) // np_
    hj = jax.lax.broadcasted_iota(jnp.int32, (H * np_, H), 1)
    ones_bd = (hi == hj).astype(BF)
    xi = jax.lax.broadcasted_iota(jnp.int32, (H, D), 0)
    xj = jax.lax.broadcasted_iota(jnp.int32, (H, D), 1) // DH
    exp8 = (xi == xj).astype(F32)
    return mask01, ones_bd, exp8


def _moa_layer(x, bounds, np_, masks, gW, gb, WqA, WkAT, WvA, WoS,
               W1, b1, W2, b2):
    mask01, ones_bd, exp8 = masks
    xb = x.astype(BF)
    xT = x.T.astype(BF)

    xr = jnp.concatenate([x[p:p + 1, :] for p in
                          [b - 1 for (_, b) in bounds]], axis=0)
    gl = jnp.dot(xr, gW, preferred_element_type=F32) + gb
    gl = gl - jnp.max(gl, axis=-1, keepdims=True)
    ge = jnp.exp(gl)
    gates = ge / jnp.sum(ge, axis=-1, keepdims=True)
    gparts = []
    for si, (s0, s1) in enumerate(bounds):
        gparts.append(jnp.broadcast_to(gates[si:si + 1, :], (s1 - s0, E)))
    tail = np_ - bounds[-1][1]
    if tail:
        gparts.append(jnp.zeros((tail, E), F32))
    G = jnp.concatenate(gparts, axis=0)

    Q = jnp.dot(xb, WqA, preferred_element_type=F32).astype(BF)
    KT = jnp.dot(WkAT, xT, preferred_element_type=F32).astype(BF)
    V = jnp.dot(xb, WvA, preferred_element_type=F32).astype(BF)

    og_parts = []
    for e in range(E):
        p_heads = []
        for h in range(H):
            base = e * D + h * DH
            s = jnp.dot(Q[:, base:base + DH], KT[base:base + DH, :],
                        preferred_element_type=F32)
            p_heads.append((jnp.exp(s) * mask01).astype(BF))
        P = jnp.concatenate(p_heads, axis=1)
        rs = jnp.dot(P, ones_bd, preferred_element_type=F32)
        rrg = G[:, e:e + 1] / (rs + 1e-30)
        scale_e = jnp.dot(rrg, exp8, preferred_element_type=F32)
        o_heads = []
        for h in range(H):
            base = e * D + h * DH
            o_heads.append(jnp.dot(p_heads[h], V[:, base:base + DH],
                                   preferred_element_type=F32))
        o = jnp.concatenate(o_heads, axis=1)
        og_parts.append(o * scale_e)
    OG = jnp.concatenate(og_parts, axis=1).astype(BF)
    acc = jnp.dot(OG, WoS, preferred_element_type=F32)

    x1 = _ln(x + acc)
    h1 = jax.nn.gelu(jnp.dot(x1.astype(BF), W1,
                             preferred_element_type=F32) + b1)
    x2 = _ln(x1 + jnp.dot(h1.astype(BF), W2,
                          preferred_element_type=F32) + b2)
    return x2


def _net_body(xt_ref, xc_ref,
              gWt_ref, gbt_ref, WqAt_ref, WkATt_ref, WvAt_ref, WoSt_ref,
              gWc_ref, gbc_ref, WqAc_ref, WkATc_ref, WvAc_ref, WoSc_ref,
              W1a_ref, b1a_ref, W2a_ref, b2a_ref,
              W1b_ref, b1b_ref, W2b_ref, b2b_ref,
              clfW_ref, clfb_ref, out_ref):
    masks_t = _masks(BOUNDS_T, NP_T)
    masks_c = _masks(BOUNDS_C, NP_C)
    wt = (gWt_ref[...], gbt_ref[0], WqAt_ref[...], WkATt_ref[...],
          WvAt_ref[...], WoSt_ref[...])
    wc = (gWc_ref[...], gbc_ref[0], WqAc_ref[...], WkATc_ref[...],
          WvAc_ref[...], WoSc_ref[...])
    ffn1 = (W1a_ref[...], b1a_ref[0], W2a_ref[...], b2a_ref[0])
    ffn2 = (W1b_ref[...], b1b_ref[0], W2b_ref[...], b2b_ref[0])

    xt = xt_ref[0]
    xc = xc_ref[0]
    xt = _moa_layer(xt, BOUNDS_T, NP_T, masks_t, *wt, *ffn1)
    xc = _moa_layer(xc, BOUNDS_C, NP_C, masks_c, *wc, *ffn1)
    xt = _moa_layer(xt, BOUNDS_T, NP_T, masks_t, *wt, *ffn2)
    xc = _moa_layer(xc, BOUNDS_C, NP_C, masks_c, *wc, *ffn2)

    rt = jnp.concatenate([xt[p:p + 1, :] for p in ROUTERS_T], axis=0)
    rc = jnp.concatenate([xc[p:p + 1, :] for p in ROUTERS_C], axis=0)
    t_repr = jnp.mean(_ln(rt), axis=0, keepdims=True)
    c_repr = jnp.mean(_ln(rc), axis=0, keepdims=True)
    final = jax.nn.gelu(jnp.concatenate([t_repr, c_repr], axis=1))
    out_ref[0] = (jnp.dot(final, clfW_ref[...],
                          preferred_element_type=F32) + clfb_ref[0])


def _whole(shape):
    nd = len(shape)
    return pl.BlockSpec(shape, lambda b: (0,) * nd)


@jax.jit
def _net(Xt, Xc, wt, wc, ffn1, ffn2, clfW, clfb):
    args = (Xt, Xc) + wt + wc + ffn1 + ffn2 + (clfW, clfb)
    in_specs = [
        pl.BlockSpec((1, NP_T, D), lambda b: (b, 0, 0)),
        pl.BlockSpec((1, NP_C, D), lambda b: (b, 0, 0)),
    ] + [_whole(a.shape) for a in args[2:]]
    return pl.pallas_call(
        _net_body,
        grid=(B,),
        in_specs=in_specs,
        out_specs=pl.BlockSpec((1, 1, NC), lambda b: (b, 0, 0)),
        out_shape=jax.ShapeDtypeStruct((B, 1, NC), jnp.float32),
    )(*args).reshape(B, NC)


def _pack_branch(p, br):
    Wq = p['Wq_' + br]; Wk = p['Wk_' + br]; Wv = p['Wv_' + br]
    Wo = p['Wo_' + br]
    scale = 1.0 / (DH ** 0.5)
    WqA = (jnp.transpose(Wq, (1, 0, 2)).reshape(D, ED) * scale).astype(BF)
    WkAT = jnp.transpose(Wk, (0, 2, 1)).reshape(ED, D).astype(BF)
    WvA = jnp.transpose(Wv, (1, 0, 2)).reshape(D, ED).astype(BF)
    WoS = Wo.reshape(ED, D).astype(BF)
    return (p['gate_W_' + br], p['gate_b_' + br].reshape(1, E),
            WqA, WkAT, WvA, WoS)


def kernel(x_enc, x_mark_enc, x_dec, x_mark_dec, params):
    p = params

    xs_t = []
    for gi, patch in enumerate(PATCH):
        pad = jnp.repeat(x_enc[:, -1:, :], patch, axis=1)
        xp = jnp.concatenate([x_enc, pad], axis=1)
        n = xp.shape[1] // patch
        tok = xp.reshape(B, n, patch * C) @ p['emb_t_W'][gi] + p['emb_t_b'][gi]
        r = jnp.broadcast_to(p['router_t'][gi][None, None, :], (B, 1, D))
        xs_t.append(jnp.concatenate([tok, r], axis=1) + p['pos_t'][gi][None])
    xs_c = []
    for si, u in enumerate(UPD):
        xc = jnp.einsum('blc,cu->bul', x_enc, p['up_W'][si])
        tok = xc @ p['emb_c_W'][si] + p['emb_c_b'][si]
        r = jnp.broadcast_to(p['router_c'][si][None, None, :], (B, 1, D))
        xs_c.append(jnp.concatenate([tok, r], axis=1) + p['pos_c'][si][None])

    Xt = jnp.concatenate(
        xs_t + [jnp.zeros((B, NP_T - sum(NS_T), D), F32)], axis=1)
    Xc = jnp.concatenate(
        xs_c + [jnp.zeros((B, NP_C - sum(NS_C), D), F32)], axis=1)

    ffn = []
    for l in range(NL):
        ffn.append((p['ffn_W1'][l].astype(BF), p['ffn_b1'][l].reshape(1, DFF),
                    p['ffn_W2'][l].astype(BF), p['ffn_b2'][l].reshape(1, D)))

    return _net(Xt, Xc, _pack_branch(p, 't'), _pack_branch(p, 'c'),
                ffn[0], ffn[1], p['clf_W'], p['clf_b'].reshape(1, NC))


# speedup vs baseline: 3.7307x; 1.0664x over previous
import functools

import jax
import jax.numpy as jnp
from jax.experimental import pallas as pl
from jax.experimental.pallas import tpu as pltpu

B = 16; L = 512; C = 19; D = 256; H = 8; E = 8; NL = 2; DFF = 512; NC = 4
DH = D // H
ED = E * D
PATCH = [8, 16, 32]
UPD = [19, 38]

NS_T = [L // p + 2 for p in PATCH]
NS_C = [u + 1 for u in UPD]
NTOK_T = [L // p + 1 for p in PATCH]
NQ_T = 120
NQ_C = 64
KP = 128

BF = jnp.bfloat16
F32 = jnp.float32


def _bounds(ns):
    out, s = [], 0
    for n in ns:
        out.append((s, s + n))
        s += n
    return out


BOUNDS_T = _bounds(NS_T)
BOUNDS_C = _bounds(NS_C)
ROUTERS_T = [b - 1 for (_, b) in BOUNDS_T]
ROUTERS_C = [b - 1 for (_, b) in BOUNDS_C]


def _ln(x):
    m = jnp.mean(x, axis=-1, keepdims=True)
    v = jnp.mean((x - m) ** 2, axis=-1, keepdims=True)
    return (x - m) * jax.lax.rsqrt(v + 1e-5)


def _masks(bounds, nq):
    ri = jax.lax.broadcasted_iota(jnp.int32, (nq, KP), 0)
    ci = jax.lax.broadcasted_iota(jnp.int32, (nq, KP), 1)
    mask = jnp.zeros((nq, KP), jnp.bool_)
    for (s0, s1) in bounds:
        mask = mask | ((ri >= s0) & (ri < s1) & (ci >= s0) & (ci < s1))
    mask01 = mask.astype(F32)
    hi = jax.lax.broadcasted_iota(jnp.int32, (H * KP, H), 0) // KP
    hj = jax.lax.broadcasted_iota(jnp.int32, (H * KP, H), 1)
    ones_bd = (hi == hj).astype(BF)
    xi = jax.lax.broadcasted_iota(jnp.int32, (H, D), 0)
    xj = jax.lax.broadcasted_iota(jnp.int32, (H, D), 1) // DH
    exp8 = (xi == xj).astype(F32)
    return mask01, ones_bd, exp8


def _moa_layer(x, bounds, nq, masks, gW, gb, WqA, WkAT, WvA, WoS,
               W1, b1, W2, b2):
    mask01, ones_bd, exp8 = masks
    xq = x[:nq]
    xqb = xq.astype(BF)
    xT = x.T.astype(BF)

    xr = jnp.concatenate([x[p:p + 1, :] for p in
                          [b - 1 for (_, b) in bounds]], axis=0)
    gl = jnp.dot(xr, gW, preferred_element_type=F32) + gb
    gl = gl - jnp.max(gl, axis=-1, keepdims=True)
    ge = jnp.exp(gl)
    gates = ge / jnp.sum(ge, axis=-1, keepdims=True)
    gparts = []
    for si, (s0, s1) in enumerate(bounds):
        gparts.append(jnp.broadcast_to(gates[si:si + 1, :], (s1 - s0, E)))
    tail = nq - bounds[-1][1]
    if tail:
        gparts.append(jnp.zeros((tail, E), F32))
    G = jnp.concatenate(gparts, axis=0)

    Q = jnp.dot(xqb, WqA, preferred_element_type=F32).astype(BF)
    KT = jnp.dot(WkAT, xT, preferred_element_type=F32).astype(BF)
    V = jnp.dot(x.astype(BF), WvA,
                preferred_element_type=F32).astype(BF)

    og_parts = []
    for e in range(E):
        p_heads = []
        for h in range(H):
            base = e * D + h * DH
            s = jnp.dot(Q[:, base:base + DH], KT[base:base + DH, :],
                        preferred_element_type=F32)
            p_heads.append((jnp.exp(s) * mask01).astype(BF))
        P = jnp.concatenate(p_heads, axis=1)
        rs = jnp.dot(P, ones_bd, preferred_element_type=F32)
        rrg = G[:, e:e + 1] / (rs + 1e-30)
        scale_e = jnp.dot(rrg, exp8, preferred_element_type=F32)
        o_heads = []
        for h in range(H):
            base = e * D + h * DH
            o_heads.append(jnp.dot(p_heads[h], V[:, base:base + DH],
                                   preferred_element_type=F32))
        o = jnp.concatenate(o_heads, axis=1)
        og_parts.append(o * scale_e)
    OG = jnp.concatenate(og_parts, axis=1).astype(BF)
    acc = jnp.dot(OG, WoS, preferred_element_type=F32)

    x1 = _ln(xq + acc)
    h1 = jax.nn.gelu(jnp.dot(x1.astype(BF), W1,
                             preferred_element_type=F32) + b1)
    x2 = _ln(x1 + jnp.dot(h1.astype(BF), W2,
                          preferred_element_type=F32) + b2)
    return x2


def _pad_kp(x, nq):
    return jnp.concatenate([x, jnp.zeros((KP - nq, D), F32)], axis=0)


def _net_body(xp8_ref, xp16_ref, xp32_ref, xeT_ref,
              We1_ref, We2_ref, We3_ref, pr1_ref, pr2_ref, pr3_ref,
              uW1_ref, uW2_ref, Wc1_ref, Wc2_ref, prc1_ref, prc2_ref,
              gWt_ref, gbt_ref, WqAt_ref, WkATt_ref, WvAt_ref, WoSt_ref,
              gWc_ref, gbc_ref, WqAc_ref, WkATc_ref, WvAc_ref, WoSc_ref,
              W1a_ref, b1a_ref, W2a_ref, b2a_ref,
              W1b_ref, b1b_ref, W2b_ref, b2b_ref,
              clfW_ref, clfb_ref, out_ref):
    masks_t = _masks(BOUNDS_T, NQ_T)
    masks_c = _masks(BOUNDS_C, NQ_C)
    wt = (gWt_ref[...], gbt_ref[0], WqAt_ref[...], WkATt_ref[...],
          WvAt_ref[...], WoSt_ref[...])
    wc = (gWc_ref[...], gbc_ref[0], WqAc_ref[...], WkATc_ref[...],
          WvAc_ref[...], WoSc_ref[...])
    ffn1 = (W1a_ref[...], b1a_ref[0], W2a_ref[...], b2a_ref[0])
    ffn2 = (W1b_ref[...], b1b_ref[0], W2b_ref[...], b2b_ref[0])

    streams_t = []
    for xp_ref, We_ref, pr_ref, ntok in (
            (xp8_ref, We1_ref, pr1_ref, NTOK_T[0]),
            (xp16_ref, We2_ref, pr2_ref, NTOK_T[1]),
            (xp32_ref, We3_ref, pr3_ref, NTOK_T[2])):
        tok = jnp.dot(xp_ref[0], We_ref[...], preferred_element_type=F32)
        tok = jnp.concatenate([tok, jnp.zeros((1, D), F32)], axis=0)
        streams_t.append(tok + pr_ref[...])
    xt = _pad_kp(jnp.concatenate(streams_t, axis=0), sum(NS_T))

    xeT = xeT_ref[0]
    streams_c = []
    for uW_ref, Wc_ref, prc_ref, u in ((uW1_ref, Wc1_ref, prc1_ref, UPD[0]),
                                       (uW2_ref, Wc2_ref, prc2_ref, UPD[1])):
        xc_ = jnp.dot(uW_ref[...], xeT, preferred_element_type=F32)
        tok = jnp.dot(xc_.astype(BF), Wc_ref[...], preferred_element_type=F32)
        tok = jnp.concatenate([tok, jnp.zeros((1, D), F32)], axis=0)
        streams_c.append(tok + prc_ref[...])
    xc = _pad_kp(jnp.concatenate(streams_c, axis=0), sum(NS_C))

    xt = _pad_kp(_moa_layer(xt, BOUNDS_T, NQ_T, masks_t, *wt, *ffn1), NQ_T)
    xc = _pad_kp(_moa_layer(xc, BOUNDS_C, NQ_C, masks_c, *wc, *ffn1), NQ_C)
    xt = _moa_layer(xt, BOUNDS_T, NQ_T, masks_t, *wt, *ffn2)
    xc = _moa_layer(xc, BOUNDS_C, NQ_C, masks_c, *wc, *ffn2)

    rt = jnp.concatenate([xt[p:p + 1, :] for p in ROUTERS_T], axis=0)
    rc = jnp.concatenate([xc[p:p + 1, :] for p in ROUTERS_C], axis=0)
    t_repr = jnp.mean(_ln(rt), axis=0, keepdims=True)
    c_repr = jnp.mean(_ln(rc), axis=0, keepdims=True)
    final = jax.nn.gelu(jnp.concatenate([t_repr, c_repr], axis=1))
    out_ref[0] = (jnp.dot(final, clfW_ref[...],
                          preferred_element_type=F32) + clfb_ref[0])


def _whole(shape):
    nd = len(shape)
    return pl.BlockSpec(shape, lambda b: (0,) * nd)


@jax.jit
def _net(xp8, xp16, xp32, xeT, emb_t, emb_c, wt, wc, ffn1, ffn2, clfW, clfb):
    args = ((xp8, xp16, xp32, xeT) + emb_t + emb_c + wt + wc + ffn1 + ffn2
            + (clfW, clfb))
    in_specs = [
        pl.BlockSpec((1,) + xp8.shape[1:], lambda b: (b, 0, 0)),
        pl.BlockSpec((1,) + xp16.shape[1:], lambda b: (b, 0, 0)),
        pl.BlockSpec((1,) + xp32.shape[1:], lambda b: (b, 0, 0)),
        pl.BlockSpec((1,) + xeT.shape[1:], lambda b: (b, 0, 0)),
    ] + [_whole(a.shape) for a in args[4:]]
    return pl.pallas_call(
        _net_body,
        grid=(B,),
        in_specs=in_specs,
        out_specs=pl.BlockSpec((1, 1, NC), lambda b: (b, 0, 0)),
        out_shape=jax.ShapeDtypeStruct((B, 1, NC), jnp.float32),
    )(*args).reshape(B, NC)


def _pack_branch(p, br):
    Wq = p['Wq_' + br]; Wk = p['Wk_' + br]; Wv = p['Wv_' + br]
    Wo = p['Wo_' + br]
    scale = 1.0 / (DH ** 0.5)
    WqA = (jnp.transpose(Wq, (1, 0, 2)).reshape(D, ED) * scale).astype(BF)
    WkAT = jnp.transpose(Wk, (0, 2, 1)).reshape(ED, D).astype(BF)
    WvA = jnp.transpose(Wv, (1, 0, 2)).reshape(D, ED).astype(BF)
    WoS = Wo.reshape(ED, D).astype(BF)
    return (p['gate_W_' + br], p['gate_b_' + br].reshape(1, E),
            WqA, WkAT, WvA, WoS)


def kernel(x_enc, x_mark_enc, x_dec, x_mark_dec, params):
    p = params

    xps = []
    for patch in PATCH:
        pad = jnp.repeat(x_enc[:, -1:, :], patch, axis=1)
        xp = jnp.concatenate([x_enc, pad], axis=1)
        n = xp.shape[1] // patch
        xps.append(xp.reshape(B, n, patch * C).astype(BF))
    xeT = jnp.transpose(x_enc, (0, 2, 1)).astype(BF)

    emb_t = tuple(p['emb_t_W'][gi].astype(BF) for gi in range(3))
    prs = []
    for gi in range(3):
        body = jnp.broadcast_to(p['emb_t_b'][gi][None, :], (NTOK_T[gi], D))
        last = p['router_t'][gi][None, :]
        prs.append(p['pos_t'][gi] + jnp.concatenate([body, last], axis=0))
    emb_t = emb_t + tuple(prs)

    emb_c = tuple(p['up_W'][si].T.astype(BF) for si in range(2))
    emb_c = emb_c + tuple(p['emb_c_W'][si].astype(BF) for si in range(2))
    prcs = []
    for si in range(2):
        body = jnp.broadcast_to(p['emb_c_b'][si][None, :], (UPD[si], D))
        last = p['router_c'][si][None, :]
        prcs.append(p['pos_c'][si] + jnp.concatenate([body, last], axis=0))
    emb_c = emb_c + tuple(prcs)

    ffn = []
    for l in range(NL):
        ffn.append((p['ffn_W1'][l].astype(BF), p['ffn_b1'][l].reshape(1, DFF),
                    p['ffn_W2'][l].astype(BF), p['ffn_b2'][l].reshape(1, D)))

    return _net(xps[0], xps[1], xps[2], xeT, emb_t, emb_c,
                _pack_branch(p, 't'), _pack_branch(p, 'c'),
                ffn[0], ffn[1], p['clf_W'], p['clf_b'].reshape(1, NC))
